# Initial kernel scaffold; baseline (speedup 1.0000x reference)
import jax
import jax.numpy as jnp
from jax.experimental import pallas as pl


def kernel(z):
    raise NotImplementedError("write your pallas kernel here")


# speedup vs baseline: 1.0145x; 1.0145x over previous
import functools

import jax
import jax.numpy as jnp
from jax import lax
from jax.experimental import pallas as pl
from jax.experimental.pallas import tpu as pltpu

_N_SAMPLE = 8
_KS0 = 0
_KS1 = 42
_KS2 = _KS0 ^ _KS1 ^ 0x1BD11BDA
_ROT_A = (13, 15, 26, 6)
_ROT_B = (17, 29, 16, 24)
_TINY = 1.1754943508222875e-38
_NEG_HUGE = -3.4e38
_IMAX = 0x7FFFFFFF


def _rotl(x, d):
    return (x << d) | lax.shift_right_logical(x, 32 - d)


def _threefry_xor(x1):
    x0 = jnp.zeros_like(x1) + _KS0
    x1 = x1 + _KS1
    sched = (
        (_ROT_B, _KS1, _KS2 + 1),
        (_ROT_A, _KS2, _KS0 + 2),
        (_ROT_B, _KS0, _KS1 + 3),
        (_ROT_A, _KS1, _KS2 + 4),
        (_ROT_B, _KS2, _KS0 + 5),
    )
    rots = _ROT_A
    for rot_next, k0, k1 in sched:
        for r in rots:
            x0 = x0 + x1
            x1 = _rotl(x1, r)
            x1 = x1 ^ x0
        x0 = x0 + k0
        x1 = x1 + k1
        rots = rot_next
    return x0 ^ x1


def _gumbel_from_bits(bits):
    fb = lax.shift_right_logical(bits, 9) | 0x3F800000
    f = lax.bitcast_convert_type(fb, jnp.float32) - jnp.float32(1.0)
    u = f + jnp.float32(_TINY)
    return -jnp.log(-jnp.log(u))


def _sample_body(z_ref, out_ref, acc_val, acc_idx, *, nchunks, B, H, CH):
    c = pl.program_id(0)

    @pl.when(c == 0)
    def _init():
        acc_val[...] = jnp.full((_N_SAMPLE, B, CH), _NEG_HUGE, jnp.float32)
        acc_idx[...] = jnp.full((_N_SAMPLE, B, CH), _IMAX, jnp.int32)

    z = z_ref[...]
    h = c * CH + lax.broadcasted_iota(jnp.int32, (B, CH), 1)
    base = lax.broadcasted_iota(jnp.int32, (B, CH), 0) * H + h
    valid = h < H
    for s in range(_N_SAMPLE):
        bits = _threefry_xor(base + s * (B * H))
        score = _gumbel_from_bits(bits) + z
        score = jnp.where(valid, score, _NEG_HUGE)
        take = score > acc_val[s]
        acc_val[s] = jnp.where(take, score, acc_val[s])
        acc_idx[s] = jnp.where(take, h, acc_idx[s])

    @pl.when(c == nchunks - 1)
    def _finalize():
        for s in range(_N_SAMPLE):
            av = acc_val[s]
            mx = jnp.max(av, axis=1, keepdims=True)
            sel = jnp.where(av == mx, acc_idx[s], _IMAX)
            out_ref[:, s : s + 1] = jnp.min(sel, axis=1, keepdims=True)


def _onehot_body(samp_ref, out_ref, *, B, BH):
    c = pl.program_id(0)
    hidx = c * BH + lax.broadcasted_iota(jnp.int32, (_N_SAMPLE, B, BH), 2)
    samp = samp_ref[...][:, :, None]
    out_ref[...] = jnp.where(hidx == samp, jnp.float32(1.0), jnp.float32(0.0))


def kernel(z):
    B, H = z.shape
    CH = 512
    nch = pl.cdiv(H, CH)
    samples_bn = pl.pallas_call(
        functools.partial(_sample_body, nchunks=nch, B=B, H=H, CH=CH),
        grid=(nch,),
        in_specs=[pl.BlockSpec((B, CH), lambda c: (0, c))],
        out_specs=pl.BlockSpec((B, _N_SAMPLE), lambda c: (0, 0)),
        out_shape=jax.ShapeDtypeStruct((B, _N_SAMPLE), jnp.int32),
        scratch_shapes=[
            pltpu.VMEM((_N_SAMPLE, B, CH), jnp.float32),
            pltpu.VMEM((_N_SAMPLE, B, CH), jnp.int32),
        ],
    )(z)
    samp = samples_bn.reshape(_N_SAMPLE, B)
    BH = 2048
    nbh = pl.cdiv(H, BH)
    return pl.pallas_call(
        functools.partial(_onehot_body, B=B, BH=BH),
        grid=(nbh,),
        in_specs=[pl.BlockSpec((_N_SAMPLE, B), lambda c: (0, 0))],
        out_specs=pl.BlockSpec((_N_SAMPLE, B, BH), lambda c: (0, 0, c)),
        out_shape=jax.ShapeDtypeStruct((_N_SAMPLE, B, H), jnp.float32),
    )(samp)
